# baseline (speedup 1.0000x reference)
import jax
import jax.numpy as jnp
from jax.experimental import pallas as pl
from jax.experimental.pallas import tpu as pltpu
from jax.experimental.pallas import tpu_sc as plsc

_VOCAB = 1000
_DIM = 64
_BATCH = 1024
_SEQ = 50
_N = _BATCH * _SEQ

_WINDOW = 256


def kernel(token_ids, W):
    idx = token_ids.reshape(1, _N)
    w_pad = jnp.pad(W, ((0, 0), (0, 128 - _DIM)))
    mesh = plsc.VectorSubcoreMesh(core_axis_name="core", subcore_axis_name="subcore")

    @pl.kernel(
        out_type=jax.ShapeDtypeStruct((_N, 128), W.dtype),
        mesh=mesh,
    )
    def gather_kernel(w_hbm, i_hbm, o_hbm):
        def body(i_vmem, o_vmem):
            pltpu.sync_copy(w_hbm.at[i_vmem.at[0]], o_vmem)

        pltpu.emit_pipeline(
            body,
            grid=(_N // _WINDOW,),
            in_specs=[pl.BlockSpec((1, _WINDOW), index_map=lambda i: (0, i))],
            out_specs=[pl.BlockSpec((_WINDOW, 128), index_map=lambda i: (i, 0))],
            core_axis_name=("core", "subcore"),
            dimension_semantics=(pltpu.PARALLEL,),
        )(i_hbm, o_hbm)

    out = gather_kernel(w_pad, idx)
    return out[:, :_DIM].reshape(_BATCH, _SEQ, _DIM)


# speedup vs baseline: 1.0090x; 1.0090x over previous
import functools

import jax
import jax.numpy as jnp
from jax import lax
from jax.experimental import pallas as pl
from jax.experimental.pallas import tpu as pltpu
from jax.experimental.pallas import tpu_sc as plsc

_VOCAB = 1000
_DIM = 64
_PAD = 128
_BATCH = 1024
_SEQ = 50
_N = _BATCH * _SEQ

_NC = 2
_NS = 16
_NW = _NC * _NS
_B_PER_W = _N // _NW
_CHUNK = 200
_NCHUNK = _B_PER_W // _CHUNK


def kernel(token_ids, W):
    idx = token_ids.reshape(_N)
    w_pad = jnp.pad(W, ((0, 0), (0, _PAD - _DIM)))
    mesh = plsc.VectorSubcoreMesh(core_axis_name="c", subcore_axis_name="s")

    @functools.partial(
        pl.kernel,
        mesh=mesh,
        out_type=jax.ShapeDtypeStruct((_N, _DIM), W.dtype),
        scratch_types=[
            pltpu.VMEM((_B_PER_W,), jnp.int32),
            pltpu.VMEM((_CHUNK, _PAD), jnp.float32),
            pltpu.VMEM((_CHUNK, _PAD), jnp.float32),
            pltpu.VMEM((_CHUNK, _DIM), jnp.float32),
            pltpu.VMEM((_CHUNK, _DIM), jnp.float32),
            pltpu.SemaphoreType.DMA,
            pltpu.SemaphoreType.DMA,
            pltpu.SemaphoreType.DMA,
            pltpu.SemaphoreType.DMA,
        ],
    )
    def gather_kernel(table_hbm, idx_hbm, out_hbm, idx_v,
                      wide_a, wide_b, nar_a, nar_b, sg_a, sg_b, sw_a, sw_b):
        wid = lax.axis_index("s") * _NC + lax.axis_index("c")
        base = wid * _B_PER_W
        pltpu.sync_copy(idx_hbm.at[pl.ds(base, _B_PER_W)], idx_v)

        wides = (wide_a, wide_b)
        nars = (nar_a, nar_b)
        sgs = (sg_a, sg_b)
        sws = (sw_a, sw_b)

        def gather_desc(c):
            b = c % 2
            return pltpu.make_async_copy(
                table_hbm.at[idx_v.at[pl.ds(c * _CHUNK, _CHUNK)]], wides[b], sgs[b]
            )

        def wb_desc(c):
            b = c % 2
            return pltpu.make_async_copy(
                nars[b], out_hbm.at[pl.ds(base + c * _CHUNK, _CHUNK)], sws[b]
            )

        gather_desc(0).start()
        for c in range(_NCHUNK):
            b = c % 2
            if c + 1 < _NCHUNK:
                gather_desc(c + 1).start()
            gather_desc(c).wait()
            if c >= 2:
                wb_desc(c - 2).wait()
            wide, nar = wides[b], nars[b]

            @pl.loop(0, _CHUNK)
            def _(r):
                for cc in range(_DIM // 16):
                    nar[r, pl.ds(cc * 16, 16)] = wide[r, pl.ds(cc * 16, 16)]

            wb_desc(c).start()
        if _NCHUNK >= 2:
            wb_desc(_NCHUNK - 2).wait()
        wb_desc(_NCHUNK - 1).wait()

    out = gather_kernel(w_pad, idx)
    return out.reshape(_BATCH, _SEQ, _DIM)


# speedup vs baseline: 1.6388x; 1.6242x over previous
import dataclasses
import functools

import jax
import jax.numpy as jnp
from jax import lax
from jax.experimental import pallas as pl
from jax.experimental.pallas import tpu as pltpu
from jax.experimental.pallas import tpu_sc as plsc

_VOCAB = 1000
_DIM = 64
_BATCH = 1024
_SEQ = 50

_NC = 2
_NS = 16
_DSLAB = 8
_NSLAB = _DIM // _DSLAB
_SGRP = 13
_LANES = 16


def kernel(token_ids, W):
    tok_t = token_ids.T.reshape(_SEQ, 1, _BATCH)
    w_t = W.T.reshape(_DIM, 1, _VOCAB)
    mesh = plsc.VectorSubcoreMesh(core_axis_name="c", subcore_axis_name="s")
    cp = pltpu.CompilerParams()
    if "needs_layout_passes" in pltpu.CompilerParams.__dataclass_fields__:
        cp = dataclasses.replace(cp, needs_layout_passes=False)

    @functools.partial(
        pl.kernel,
        mesh=mesh,
        compiler_params=cp,
        out_type=jax.ShapeDtypeStruct((_SEQ, _DIM, _BATCH), W.dtype),
        scratch_types=[
            pltpu.VMEM((_DSLAB, 1, _VOCAB), jnp.float32),
            pltpu.VMEM((_SGRP, 1, _BATCH), jnp.int32),
            pltpu.VMEM((1, _DSLAB, _BATCH), jnp.float32),
            pltpu.VMEM((1, _DSLAB, _BATCH), jnp.float32),
            pltpu.SemaphoreType.DMA,
            pltpu.SemaphoreType.DMA,
        ],
    )
    def emb_kernel(w_hbm, tok_hbm, out_hbm, wbuf, tokbuf, ob_a, ob_b, sw_a, sw_b):
        wid = lax.axis_index("s") * _NC + lax.axis_index("c")
        dslab = wid % _NSLAB
        d0 = dslab * _DSLAB
        grp = wid // _NSLAB
        s0 = jnp.minimum(grp * _SGRP, _SEQ - _SGRP)

        pltpu.sync_copy(w_hbm.at[pl.ds(d0, _DSLAB)], wbuf)
        pltpu.sync_copy(tok_hbm.at[pl.ds(s0, _SGRP)], tokbuf)

        obufs = (ob_a, ob_b)
        sws = (sw_a, sw_b)
        zvec = jnp.zeros((_LANES,), jnp.int32)
        dvecs = [jnp.full((_LANES,), d, jnp.int32) for d in range(_DSLAB)]

        def compute(si, ob):
            @pl.loop(0, _BATCH // _LANES)
            def _(bg):
                b0 = bg * _LANES
                tv = tokbuf[si, 0, pl.ds(b0, _LANES)]
                for d in range(_DSLAB):
                    vals = plsc.load_gather(wbuf, [dvecs[d], zvec, tv])
                    ob[0, d, pl.ds(b0, _LANES)] = vals

        def wb_desc(si, b):
            return pltpu.make_async_copy(
                obufs[b],
                out_hbm.at[pl.ds(s0 + si, 1), pl.ds(d0, _DSLAB)],
                sws[b],
            )

        for si in range(_SGRP):
            b = si % 2
            if si >= 2:
                wb_desc(si - 2, b).wait()
            compute(si, obufs[b])
            wb_desc(si, b).start()
        wb_desc(_SGRP - 2, (_SGRP - 2) % 2).wait()
        wb_desc(_SGRP - 1, (_SGRP - 1) % 2).wait()

    out = emb_kernel(w_t, tok_t)
    return out.transpose(2, 0, 1)
